# Initial kernel scaffold; baseline (speedup 1.0000x reference)
import jax
import jax.numpy as jnp
from jax.experimental import pallas as pl


def kernel(pointcloud, sa1_w0, sa1_g0, sa1_b0, sa1_w1, sa1_g1, sa1_b1, sa1_w2, sa1_g2, sa1_b2, sa2_w0, sa2_g0, sa2_b0, sa2_w1, sa2_g1, sa2_b1, sa2_w2, sa2_g2, sa2_b2, sa3_w0, sa3_g0, sa3_b0, sa3_w1, sa3_g1, sa3_b1, sa3_w2, sa3_g2, sa3_b2, fc0_w, fc0_b, fc1_w, fc1_b, fc2_w, fc2_b, bn0_g, bn0_b, bn1_g, bn1_b):
    raise NotImplementedError("write your pallas kernel here")


# speedup vs baseline: 11.5055x; 11.5055x over previous
import functools

import numpy as np
import jax
import jax.numpy as jnp
from jax.experimental import pallas as pl

_call = pl.pallas_call

_SQ = np.float32(np.sqrt(np.float32(1.0 + 1e-5)))
_HI = jax.lax.Precision.HIGHEST


def _fps_kernel(pts_ref, cent_ref, *, n, npoint):
    x = pts_ref[0]
    y = pts_ref[1]
    z = pts_ref[2]
    b = x.shape[0]
    iota = jax.lax.broadcasted_iota(jnp.int32, (b, n), 1)
    iota_p = jax.lax.broadcasted_iota(jnp.int32, (b, npoint), 1)
    xs = x[:, 0:1]
    ys = y[:, 0:1]
    zs = z[:, 0:1]
    cx = jnp.where(iota_p == 0, xs, 0.0)
    cy = jnp.where(iota_p == 0, ys, 0.0)
    cz = jnp.where(iota_p == 0, zs, 0.0)
    dists0 = jnp.full((b, n), 1e10, jnp.float32)

    def step(t, carry):
        dists, xs, ys, zs, cx, cy, cz = carry
        dx = x - xs
        dy = y - ys
        dz = z - zs
        d = dx * dx + dy * dy + dz * dz
        dists = jnp.minimum(dists, d)
        mx = jnp.max(dists, axis=1, keepdims=True)
        nxt = jnp.min(jnp.where(dists == mx, iota, n), axis=1, keepdims=True)
        sel = iota == nxt
        xs = jnp.sum(jnp.where(sel, x, 0.0), axis=1, keepdims=True)
        ys = jnp.sum(jnp.where(sel, y, 0.0), axis=1, keepdims=True)
        zs = jnp.sum(jnp.where(sel, z, 0.0), axis=1, keepdims=True)
        sel_p = iota_p == t
        cx = jnp.where(sel_p, xs, cx)
        cy = jnp.where(sel_p, ys, cy)
        cz = jnp.where(sel_p, zs, cz)
        return dists, xs, ys, zs, cx, cy, cz

    _, _, _, _, cx, cy, cz = jax.lax.fori_loop(
        1, npoint, step, (dists0, xs, ys, zs, cx, cy, cz), unroll=False)
    cent_ref[0] = cx
    cent_ref[1] = cy
    cent_ref[2] = cz


def _fps(coordsT, npoint):
    three, b, n = coordsT.shape
    return _call(
        functools.partial(_fps_kernel, n=n, npoint=npoint),
        out_shape=jax.ShapeDtypeStruct((3, b, npoint), jnp.float32),
    )(coordsT)


def _sa_kernel(pts_ref, p_ref, cent_ref, u_ref,
               w0_ref, g0_ref, b0_ref, w1_ref, g1_ref, b1_ref,
               w2_ref, g2_ref, b2_ref, out_ref,
               *, n, sc, ns, r2, nf):
    cent = cent_ref[0, 0]
    cx = cent[:, 0:1]
    cy = cent[:, 1:2]
    cz = cent[:, 2:3]
    px = pts_ref[0, 0:1, :]
    py = pts_ref[0, 1:2, :]
    pz = pts_ref[0, 2:3, :]
    dx = cx - px
    dy = cy - py
    dz = cz - pz
    d2 = dx * dx + dy * dy + dz * dz
    mask = (d2 <= r2).astype(jnp.float32)
    c = jnp.dot(mask.astype(jnp.bfloat16), u_ref[...],
                preferred_element_type=jnp.float32)
    kk = (jax.lax.broadcasted_iota(jnp.int32, (sc, ns, n), 1) + 1
          ).astype(jnp.float32)
    e = ((c[:, None, :] == kk) & (mask[:, None, :] > 0.0)).astype(jnp.float32)
    e2 = jnp.reshape(e, (sc * ns, n))
    w = 3 + nf + 1
    gathered = jnp.dot(e2, p_ref[0], precision=_HI)
    g3 = jnp.reshape(gathered, (sc, ns, w))
    gamma = g3[:, :, w - 1:w]
    g3 = g3 + (1.0 - gamma) * g3[:, 0:1, :]
    gx = g3[:, :, 0:1] - cx[:, :, None]
    gy = g3[:, :, 1:2] - cy[:, :, None]
    gz = g3[:, :, 2:3] - cz[:, :, None]
    grouped = jnp.concatenate([gx, gy, gz, g3[:, :, 3:3 + nf]], axis=2)
    h = jnp.reshape(grouped, (sc * ns, 3 + nf))
    for w_r, g_r, b_r in ((w0_ref, g0_ref, b0_ref),
                          (w1_ref, g1_ref, b1_ref),
                          (w2_ref, g2_ref, b2_ref)):
        h = jnp.dot(h, w_r[...], preferred_element_type=jnp.float32)
        h = g_r[...] * h / _SQ + b_r[...]
        h = jnp.maximum(h, 0.0)
    cout = h.shape[-1]
    out_ref[0] = jnp.max(jnp.reshape(h, (sc, ns, cout)), axis=1)


def _sa_stage(coordsT, p_all, centT, layers, *, radius, ns=64, sc=32):
    three, b, n = coordsT.shape
    s = centT.shape[2]
    coordsB = jnp.transpose(coordsT, (1, 0, 2))
    centB = jnp.reshape(jnp.transpose(centT, (1, 2, 0)),
                        (b, s // sc, sc, 3))
    nf = p_all.shape[2] - 4
    cout = layers[2][0].shape[1]
    r2 = np.float32(radius * radius)
    u = (jax.lax.broadcasted_iota(jnp.int32, (n, n), 0)
         <= jax.lax.broadcasted_iota(jnp.int32, (n, n), 1)).astype(jnp.bfloat16)
    wgb = []
    specs_w = []
    for w_a, g_a, b_a in layers:
        cw = w_a.shape[1]
        wgb += [w_a, jnp.reshape(g_a, (1, cw)), jnp.reshape(b_a, (1, cw))]
        specs_w += [
            pl.BlockSpec(w_a.shape, lambda bb, cc: (0, 0)),
            pl.BlockSpec((1, cw), lambda bb, cc: (0, 0)),
            pl.BlockSpec((1, cw), lambda bb, cc: (0, 0)),
        ]
    grid = (b, s // sc)
    return _call(
        functools.partial(_sa_kernel, n=n, sc=sc, ns=ns, r2=r2, nf=nf),
        grid=grid,
        in_specs=[
            pl.BlockSpec((1, 3, n), lambda bb, cc: (bb, 0, 0)),
            pl.BlockSpec((1, n, 4 + nf), lambda bb, cc: (bb, 0, 0)),
            pl.BlockSpec((1, 1, sc, 3), lambda bb, cc: (bb, cc, 0, 0)),
            pl.BlockSpec((n, n), lambda bb, cc: (0, 0)),
        ] + specs_w,
        out_specs=pl.BlockSpec((1, sc, cout), lambda bb, cc: (bb, cc, 0)),
        out_shape=jax.ShapeDtypeStruct((b, s, cout), jnp.float32),
    )(coordsB, p_all, centB, u, *wgb)


def _tail_kernel(g_ref, w0_ref, g0_ref, b0_ref, w1_ref, g1_ref, b1_ref,
                 w2_ref, g2_ref, b2_ref,
                 f0w_ref, f0b_ref, f1w_ref, f1b_ref, f2w_ref, f2b_ref,
                 n0g_ref, n0b_ref, n1g_ref, n1b_ref, out_ref, *, b, s, cin):
    h = jnp.reshape(g_ref[...], (b * s, cin))
    for w_r, g_r, b_r in ((w0_ref, g0_ref, b0_ref),
                          (w1_ref, g1_ref, b1_ref),
                          (w2_ref, g2_ref, b2_ref)):
        h = jnp.dot(h, w_r[...], preferred_element_type=jnp.float32)
        h = g_r[...] * h / _SQ + b_r[...]
        h = jnp.maximum(h, 0.0)
    feat = jnp.max(jnp.reshape(h, (b, s, h.shape[-1])), axis=1)
    x = jnp.dot(feat, f0w_ref[...], preferred_element_type=jnp.float32)
    x = x + f0b_ref[...]
    x = n0g_ref[...] * x / _SQ + n0b_ref[...]
    x = jnp.maximum(x, 0.0)
    x = jnp.dot(x, f1w_ref[...], preferred_element_type=jnp.float32)
    x = x + f1b_ref[...]
    x = n1g_ref[...] * x / _SQ + n1b_ref[...]
    x = jnp.maximum(x, 0.0)
    x = jnp.dot(x, f2w_ref[...], preferred_element_type=jnp.float32)
    out_ref[...] = x + f2b_ref[...]


def kernel(pointcloud,
           sa1_w0, sa1_g0, sa1_b0, sa1_w1, sa1_g1, sa1_b1, sa1_w2, sa1_g2, sa1_b2,
           sa2_w0, sa2_g0, sa2_b0, sa2_w1, sa2_g1, sa2_b1, sa2_w2, sa2_g2, sa2_b2,
           sa3_w0, sa3_g0, sa3_b0, sa3_w1, sa3_g1, sa3_b1, sa3_w2, sa3_g2, sa3_b2,
           fc0_w, fc0_b, fc1_w, fc1_b, fc2_w, fc2_b,
           bn0_g, bn0_b, bn1_g, bn1_b):
    b, n, _ = pointcloud.shape
    f32 = jnp.float32
    ptsT = jnp.transpose(pointcloud, (2, 0, 1))
    xyzT = ptsT[:3]
    ones1 = jnp.ones((b, n, 1), f32)

    cent1 = _fps(xyzT, 128)
    p1 = jnp.concatenate([pointcloud, ones1], axis=-1)
    l1 = ((sa1_w0, sa1_g0, sa1_b0), (sa1_w1, sa1_g1, sa1_b1),
          (sa1_w2, sa1_g2, sa1_b2))
    feats1 = _sa_stage(xyzT, p1, cent1, l1, radius=0.1)

    cent2 = _fps(cent1, 128)
    cent1_b = jnp.transpose(cent1, (1, 2, 0))
    p2 = jnp.concatenate([cent1_b, feats1, jnp.ones((b, 128, 1), f32)],
                         axis=-1)
    l2 = ((sa2_w0, sa2_g0, sa2_b0), (sa2_w1, sa2_g1, sa2_b1),
          (sa2_w2, sa2_g2, sa2_b2))
    feats2 = _sa_stage(cent1, p2, cent2, l2, radius=0.4)

    g3 = jnp.concatenate([jnp.transpose(cent2, (1, 2, 0)), feats2],
                         axis=-1)
    tail_in = [g3, sa3_w0, jnp.reshape(sa3_g0, (1, -1)), jnp.reshape(sa3_b0, (1, -1)),
               sa3_w1, jnp.reshape(sa3_g1, (1, -1)), jnp.reshape(sa3_b1, (1, -1)),
               sa3_w2, jnp.reshape(sa3_g2, (1, -1)), jnp.reshape(sa3_b2, (1, -1)),
               fc0_w, jnp.reshape(fc0_b, (1, -1)), fc1_w, jnp.reshape(fc1_b, (1, -1)),
               fc2_w, jnp.reshape(fc2_b, (1, -1)),
               jnp.reshape(bn0_g, (1, -1)), jnp.reshape(bn0_b, (1, -1)),
               jnp.reshape(bn1_g, (1, -1)), jnp.reshape(bn1_b, (1, -1))]
    out = _call(
        functools.partial(_tail_kernel, b=b, s=128, cin=259),
        out_shape=jax.ShapeDtypeStruct((b, 1), f32),
    )(*tail_in)
    return out


# speedup vs baseline: 22.0778x; 1.9189x over previous
import functools

import numpy as np
import jax
import jax.numpy as jnp
from jax.experimental import pallas as pl

_call = pl.pallas_call

_SQ = np.float32(np.sqrt(np.float32(1.0 + 1e-5)))
_HI = jax.lax.Precision.HIGHEST


def _split2(a):
    hi = a.astype(jnp.bfloat16)
    lo = (a - hi.astype(jnp.float32)).astype(jnp.bfloat16)
    return hi, lo


def _dot3(a, b):
    ah, al = _split2(a)
    bh, bl = _split2(b)
    f32 = jnp.float32
    return (jnp.dot(ah, bh, preferred_element_type=f32)
            + jnp.dot(ah, bl, preferred_element_type=f32)
            + jnp.dot(al, bh, preferred_element_type=f32))


def _fps_kernel(pts_ref, cent_ref, *, n, npoint):
    x = pts_ref[0]
    y = pts_ref[1]
    z = pts_ref[2]
    b = x.shape[0]
    iota = jax.lax.broadcasted_iota(jnp.int32, (b, n), 1)
    iota_p = jax.lax.broadcasted_iota(jnp.int32, (b, npoint), 1)
    xs = x[:, 0:1]
    ys = y[:, 0:1]
    zs = z[:, 0:1]
    cx = jnp.where(iota_p == 0, xs, 0.0)
    cy = jnp.where(iota_p == 0, ys, 0.0)
    cz = jnp.where(iota_p == 0, zs, 0.0)
    dists0 = jnp.full((b, n), 1e10, jnp.float32)

    def step(t, carry):
        dists, xs, ys, zs, cx, cy, cz = carry
        dx = x - xs
        dy = y - ys
        dz = z - zs
        d = dx * dx + dy * dy + dz * dz
        dists = jnp.minimum(dists, d)
        mx = jnp.max(dists, axis=1, keepdims=True)
        nxt = jnp.min(jnp.where(dists == mx, iota, n), axis=1, keepdims=True)
        sel = iota == nxt
        xs = jnp.sum(jnp.where(sel, x, 0.0), axis=1, keepdims=True)
        ys = jnp.sum(jnp.where(sel, y, 0.0), axis=1, keepdims=True)
        zs = jnp.sum(jnp.where(sel, z, 0.0), axis=1, keepdims=True)
        sel_p = iota_p == t
        cx = jnp.where(sel_p, xs, cx)
        cy = jnp.where(sel_p, ys, cy)
        cz = jnp.where(sel_p, zs, cz)
        return dists, xs, ys, zs, cx, cy, cz

    _, _, _, _, cx, cy, cz = jax.lax.fori_loop(
        1, npoint, step, (dists0, xs, ys, zs, cx, cy, cz), unroll=False)
    cent_ref[0] = cx
    cent_ref[1] = cy
    cent_ref[2] = cz


def _fps(coordsT, npoint):
    three, b, n = coordsT.shape
    return _call(
        functools.partial(_fps_kernel, n=n, npoint=npoint),
        out_shape=jax.ShapeDtypeStruct((3, b, npoint), jnp.float32),
    )(coordsT)


def _sa_kernel(pts_ref, p_ref, cent_ref, u_ref,
               w0_ref, g0_ref, b0_ref, w1_ref, g1_ref, b1_ref,
               w2_ref, g2_ref, b2_ref, out_ref,
               *, n, sc, ns, r2, nf):
    cent = cent_ref[0, 0]
    cx = cent[:, 0:1]
    cy = cent[:, 1:2]
    cz = cent[:, 2:3]
    px = pts_ref[0, 0:1, :]
    py = pts_ref[0, 1:2, :]
    pz = pts_ref[0, 2:3, :]
    dx = cx - px
    dy = cy - py
    dz = cz - pz
    d2 = dx * dx + dy * dy + dz * dz
    mask = (d2 <= r2).astype(jnp.float32)
    c = jnp.dot(mask.astype(jnp.bfloat16), u_ref[...],
                preferred_element_type=jnp.float32)
    kk = (jax.lax.broadcasted_iota(jnp.int32, (sc, ns, n), 1) + 1
          ).astype(jnp.float32)
    e = ((c[:, None, :] == kk) & (mask[:, None, :] > 0.0)).astype(jnp.bfloat16)
    e2 = jnp.reshape(e, (sc * ns, n))
    w = 3 + nf + 1
    ph, plo = _split2(p_ref[0])
    gathered = (jnp.dot(e2, ph, preferred_element_type=jnp.float32)
                + jnp.dot(e2, plo, preferred_element_type=jnp.float32))
    g3 = jnp.reshape(gathered, (sc, ns, w))
    gamma = g3[:, :, w - 1:w]
    g3 = g3 + (1.0 - gamma) * g3[:, 0:1, :]
    gx = g3[:, :, 0:1] - cx[:, :, None]
    gy = g3[:, :, 1:2] - cy[:, :, None]
    gz = g3[:, :, 2:3] - cz[:, :, None]
    grouped = jnp.concatenate([gx, gy, gz, g3[:, :, 3:3 + nf]], axis=2)
    h = jnp.reshape(grouped, (sc * ns, 3 + nf))
    for w_r, g_r, b_r in ((w0_ref, g0_ref, b0_ref),
                          (w1_ref, g1_ref, b1_ref),
                          (w2_ref, g2_ref, b2_ref)):
        h = jnp.dot(h, w_r[...], preferred_element_type=jnp.float32)
        h = g_r[...] * h / _SQ + b_r[...]
        h = jnp.maximum(h, 0.0)
    cout = h.shape[-1]
    out_ref[0] = jnp.max(jnp.reshape(h, (sc, ns, cout)), axis=1)


def _sa_stage(coordsT, p_all, centT, layers, *, radius, ns=64, sc=32):
    three, b, n = coordsT.shape
    s = centT.shape[2]
    coordsB = jnp.transpose(coordsT, (1, 0, 2))
    centB = jnp.reshape(jnp.transpose(centT, (1, 2, 0)),
                        (b, s // sc, sc, 3))
    nf = p_all.shape[2] - 4
    cout = layers[2][0].shape[1]
    r2 = np.float32(radius * radius)
    u = (jax.lax.broadcasted_iota(jnp.int32, (n, n), 0)
         <= jax.lax.broadcasted_iota(jnp.int32, (n, n), 1)).astype(jnp.bfloat16)
    wgb = []
    specs_w = []
    for w_a, g_a, b_a in layers:
        cw = w_a.shape[1]
        wgb += [w_a, jnp.reshape(g_a, (1, cw)), jnp.reshape(b_a, (1, cw))]
        specs_w += [
            pl.BlockSpec(w_a.shape, lambda bb, cc: (0, 0)),
            pl.BlockSpec((1, cw), lambda bb, cc: (0, 0)),
            pl.BlockSpec((1, cw), lambda bb, cc: (0, 0)),
        ]
    grid = (b, s // sc)
    return _call(
        functools.partial(_sa_kernel, n=n, sc=sc, ns=ns, r2=r2, nf=nf),
        grid=grid,
        in_specs=[
            pl.BlockSpec((1, 3, n), lambda bb, cc: (bb, 0, 0)),
            pl.BlockSpec((1, n, 4 + nf), lambda bb, cc: (bb, 0, 0)),
            pl.BlockSpec((1, 1, sc, 3), lambda bb, cc: (bb, cc, 0, 0)),
            pl.BlockSpec((n, n), lambda bb, cc: (0, 0)),
        ] + specs_w,
        out_specs=pl.BlockSpec((1, sc, cout), lambda bb, cc: (bb, cc, 0)),
        out_shape=jax.ShapeDtypeStruct((b, s, cout), jnp.float32),
    )(coordsB, p_all, centB, u, *wgb)


def _tail_kernel(g_ref, w0_ref, g0_ref, b0_ref, w1_ref, g1_ref, b1_ref,
                 w2_ref, g2_ref, b2_ref,
                 f0w_ref, f0b_ref, f1w_ref, f1b_ref, f2w_ref, f2b_ref,
                 n0g_ref, n0b_ref, n1g_ref, n1b_ref, out_ref, *, b, s, cin):
    h = jnp.reshape(g_ref[...], (b * s, cin))
    for w_r, g_r, b_r in ((w0_ref, g0_ref, b0_ref),
                          (w1_ref, g1_ref, b1_ref),
                          (w2_ref, g2_ref, b2_ref)):
        h = jnp.dot(h, w_r[...], preferred_element_type=jnp.float32)
        h = g_r[...] * h / _SQ + b_r[...]
        h = jnp.maximum(h, 0.0)
    feat = jnp.max(jnp.reshape(h, (b, s, h.shape[-1])), axis=1)
    x = jnp.dot(feat, f0w_ref[...], preferred_element_type=jnp.float32) + f0b_ref[...]
    x = n0g_ref[...] * x / _SQ + n0b_ref[...]
    x = jnp.maximum(x, 0.0)
    x = jnp.dot(x, f1w_ref[...], preferred_element_type=jnp.float32) + f1b_ref[...]
    x = n1g_ref[...] * x / _SQ + n1b_ref[...]
    x = jnp.maximum(x, 0.0)
    out_ref[...] = jnp.dot(x, f2w_ref[...], preferred_element_type=jnp.float32) + f2b_ref[...]


def kernel(pointcloud,
           sa1_w0, sa1_g0, sa1_b0, sa1_w1, sa1_g1, sa1_b1, sa1_w2, sa1_g2, sa1_b2,
           sa2_w0, sa2_g0, sa2_b0, sa2_w1, sa2_g1, sa2_b1, sa2_w2, sa2_g2, sa2_b2,
           sa3_w0, sa3_g0, sa3_b0, sa3_w1, sa3_g1, sa3_b1, sa3_w2, sa3_g2, sa3_b2,
           fc0_w, fc0_b, fc1_w, fc1_b, fc2_w, fc2_b,
           bn0_g, bn0_b, bn1_g, bn1_b):
    b, n, _ = pointcloud.shape
    f32 = jnp.float32
    ptsT = jnp.transpose(pointcloud, (2, 0, 1))
    xyzT = ptsT[:3]
    ones1 = jnp.ones((b, n, 1), f32)

    cent1 = _fps(xyzT, 128)
    p1 = jnp.concatenate([pointcloud, ones1], axis=-1)
    l1 = ((sa1_w0, sa1_g0, sa1_b0), (sa1_w1, sa1_g1, sa1_b1),
          (sa1_w2, sa1_g2, sa1_b2))
    feats1 = _sa_stage(xyzT, p1, cent1, l1, radius=0.1)

    cent2 = _fps(cent1, 128)
    cent1_b = jnp.transpose(cent1, (1, 2, 0))
    p2 = jnp.concatenate([cent1_b, feats1, jnp.ones((b, 128, 1), f32)],
                         axis=-1)
    l2 = ((sa2_w0, sa2_g0, sa2_b0), (sa2_w1, sa2_g1, sa2_b1),
          (sa2_w2, sa2_g2, sa2_b2))
    feats2 = _sa_stage(cent1, p2, cent2, l2, radius=0.4)

    g3 = jnp.concatenate([jnp.transpose(cent2, (1, 2, 0)), feats2],
                         axis=-1)
    tail_in = [g3, sa3_w0, jnp.reshape(sa3_g0, (1, -1)), jnp.reshape(sa3_b0, (1, -1)),
               sa3_w1, jnp.reshape(sa3_g1, (1, -1)), jnp.reshape(sa3_b1, (1, -1)),
               sa3_w2, jnp.reshape(sa3_g2, (1, -1)), jnp.reshape(sa3_b2, (1, -1)),
               fc0_w, jnp.reshape(fc0_b, (1, -1)), fc1_w, jnp.reshape(fc1_b, (1, -1)),
               fc2_w, jnp.reshape(fc2_b, (1, -1)),
               jnp.reshape(bn0_g, (1, -1)), jnp.reshape(bn0_b, (1, -1)),
               jnp.reshape(bn1_g, (1, -1)), jnp.reshape(bn1_b, (1, -1))]
    out = _call(
        functools.partial(_tail_kernel, b=b, s=128, cin=259),
        out_shape=jax.ShapeDtypeStruct((b, 1), f32),
    )(*tail_in)
    return out
